# Initial kernel scaffold; baseline (speedup 1.0000x reference)
import jax
import jax.numpy as jnp
from jax.experimental import pallas as pl


def kernel(x, edge_index_e0, edge_index_e1, edge_index_e2, W_e0, b_e0, W_e1, b_e1, W_e2, b_e2):
    raise NotImplementedError("write your pallas kernel here")


# speedup vs baseline: 2.1239x; 2.1239x over previous
import functools

import jax
import jax.numpy as jnp
from jax import lax
from jax.experimental import pallas as pl
from jax.experimental.pallas import tpu as pltpu
from jax.experimental.pallas import tpu_sc as plsc

NC = 2
NS = 16
LANES = 16

WIN = 128
QW = 32
NQ = 4

_mesh = plsc.VectorSubcoreMesh(core_axis_name="c", subcore_axis_name="s")
_sc_params = pltpu.CompilerParams(use_tc_tiling_on_sc=False)


def _plan(n, e):
    stripe = ((n + NS - 1) // NS + 7) // 8 * 8
    n_pad = stripe * NS
    nfull = e // WIN
    rem = e - nfull * WIN
    nwin_per_tile = (nfull + NS - 1) // NS
    return stripe, n_pad, nfull, rem, nwin_per_tile


def _zero_acc(acc, zbuf, sem, sid, stripe):
    base = sid * stripe
    n_chunks = stripe // WIN
    tail = stripe - n_chunks * WIN
    copies = []
    for i in range(n_chunks):
        copies.append(pltpu.make_async_copy(
            zbuf, acc.at[pl.ds(base + i * WIN, WIN)], sem))
    if tail:
        copies.append(pltpu.make_async_copy(
            zbuf.at[pl.ds(0, tail)], acc.at[pl.ds(base + n_chunks * WIN, tail)], sem))
    for c in copies:
        c.start()
    for c in copies:
        c.wait()


def _write_out(acc, out_hbm, sid, stripe, n):
    last_rows = n - (NS - 1) * stripe

    @pl.when(sid < NS - 1)
    def _():
        pltpu.sync_copy(acc.at[pl.ds(sid * stripe, stripe)],
                        out_hbm.at[pl.ds(sid * stripe, stripe)])

    @pl.when(sid == NS - 1)
    def _():
        pltpu.sync_copy(acc.at[pl.ds((NS - 1) * stripe, last_rows)],
                        out_hbm.at[pl.ds((NS - 1) * stripe, last_rows)])


def _seg_job(src_hbm, dst_hbm, xq_hbm, out_hbm, acc, zbuf, idx_s, idx_d,
             idx_s_t, idx_d_t, rows, rows_t, sems, sid, plan, n):
    stripe, _, nfull, rem, nwin = plan
    _zero_acc(acc, zbuf, sems[0], sid, stripe)
    plsc.subcore_barrier()

    @pl.loop(0, nwin)
    def _(i):
        w = sid + i * NS

        @pl.when(w < nfull)
        def _():
            base_e = pl.multiple_of(w * WIN, WIN)
            c1 = pltpu.make_async_copy(src_hbm.at[pl.ds(base_e, WIN)], idx_s, sems[0])
            c2 = pltpu.make_async_copy(dst_hbm.at[pl.ds(base_e, WIN)], idx_d, sems[1])
            c1.start(); c2.start(); c1.wait()
            pltpu.async_copy(xq_hbm.at[idx_s], rows, sems[2]).wait()
            c2.wait()
            pltpu.sync_copy(rows, acc.at[idx_d], add=True)

    if rem:
        @pl.when(sid == NS - 1)
        def _():
            base_e = nfull * WIN
            pltpu.sync_copy(src_hbm.at[pl.ds(base_e, rem)], idx_s_t)
            pltpu.sync_copy(dst_hbm.at[pl.ds(base_e, rem)], idx_d_t)
            pltpu.async_copy(xq_hbm.at[idx_s_t], rows_t, sems[2]).wait()
            pltpu.sync_copy(rows_t, acc.at[idx_d_t], add=True)

    plsc.subcore_barrier()
    _write_out(acc, out_hbm, sid, stripe, n)
    plsc.subcore_barrier()


def _deg_job(dst_hbm, out_hbm, acc, zbuf, ones, ones_t, idx_d, idx_d_t, sems,
             sid, plan, n):
    stripe, _, nfull, rem, nwin = plan
    _zero_acc(acc, zbuf, sems[0], sid, stripe)
    plsc.subcore_barrier()

    @pl.loop(0, nwin)
    def _(i):
        w = sid + i * NS

        @pl.when(w < nfull)
        def _():
            base_e = pl.multiple_of(w * WIN, WIN)
            pltpu.sync_copy(dst_hbm.at[pl.ds(base_e, WIN)], idx_d)
            pltpu.sync_copy(ones, acc.at[idx_d], add=True)

    if rem:
        @pl.when(sid == NS - 1)
        def _():
            base_e = nfull * WIN
            pltpu.sync_copy(dst_hbm.at[pl.ds(base_e, rem)], idx_d_t)
            pltpu.sync_copy(ones_t, acc.at[idx_d_t], add=True)

    plsc.subcore_barrier()
    _write_out(acc, out_hbm, sid, stripe, n)
    plsc.subcore_barrier()


def _fill_rows(ref, vec):
    rows, width = ref.shape

    @pl.loop(0, rows)
    def _(i):
        for j in range(width // LANES):
            ref[i, pl.ds(j * LANES, LANES)] = vec


def _make_seg_kernel(n, e):
    plan = _plan(n, e)
    _, n_pad = plan[0], plan[1]
    out_t = [jax.ShapeDtypeStruct((n, QW), jnp.float32) for _ in range(3 * NQ)]
    scratch = [
        pltpu.VMEM_SHARED((n_pad, QW), jnp.float32),
        pltpu.VMEM((WIN, QW), jnp.float32),
        pltpu.VMEM((WIN,), jnp.int32),
        pltpu.VMEM((WIN,), jnp.int32),
        pltpu.VMEM((plan[3] or 8,), jnp.int32),
        pltpu.VMEM((plan[3] or 8,), jnp.int32),
        pltpu.VMEM((WIN, QW), jnp.float32),
        pltpu.VMEM((plan[3] or 8, QW), jnp.float32),
        pltpu.SemaphoreType.DMA,
        pltpu.SemaphoreType.DMA,
        pltpu.SemaphoreType.DMA,
    ]

    @functools.partial(pl.kernel, out_type=out_t, mesh=_mesh,
                       scratch_types=scratch, compiler_params=_sc_params)
    def seg_kernel(s0, d0, s1, d1, s2, d2, x0, x1, x2, x3, *rest):
        outs = rest[:12]
        acc, zbuf, idx_s, idx_d, idx_st, idx_dt, rows, rows_t = rest[12:20]
        sems = rest[20:23]
        cid = lax.axis_index("c")
        sid = lax.axis_index("s")
        zero_v = jnp.zeros((LANES,), jnp.float32)
        _fill_rows(zbuf, zero_v)

        srcs = (s0, s1, s2)
        dsts = (d0, d1, d2)
        xqs = (x0, x1, x2, x3)
        jobs0, jobs1 = [], []
        for ei in range(3):
            for q in range(NQ):
                (jobs0 if (ei * NQ + q) % 2 == 0 else jobs1).append((ei, q))

        def run(jobs):
            for ei, q in jobs:
                _seg_job(srcs[ei], dsts[ei], xqs[q], outs[ei * NQ + q],
                         acc, zbuf, idx_s, idx_d, idx_st, idx_dt,
                         rows, rows_t, sems, sid, plan, n)

        @pl.when(cid == 0)
        def _():
            run(jobs0)

        @pl.when(cid == 1)
        def _():
            run(jobs1)

    return seg_kernel


def _make_deg_kernel(n, e):
    plan = _plan(n, e)
    n_pad = plan[1]
    out_t = [jax.ShapeDtypeStruct((n, LANES), jnp.float32) for _ in range(3)]
    scratch = [
        pltpu.VMEM_SHARED((n_pad, LANES), jnp.float32),
        pltpu.VMEM((WIN, LANES), jnp.float32),
        pltpu.VMEM((WIN, LANES), jnp.float32),
        pltpu.VMEM((plan[3] or 8, LANES), jnp.float32),
        pltpu.VMEM((WIN,), jnp.int32),
        pltpu.VMEM((plan[3] or 8,), jnp.int32),
        pltpu.SemaphoreType.DMA,
        pltpu.SemaphoreType.DMA,
    ]

    @functools.partial(pl.kernel, out_type=out_t, mesh=_mesh,
                       scratch_types=scratch, compiler_params=_sc_params)
    def deg_kernel(d0, d1, d2, o0, o1, o2, acc, zbuf, ones, ones_t, idx_d,
                   idx_dt, *sems):
        cid = lax.axis_index("c")
        sid = lax.axis_index("s")
        zero_v = jnp.zeros((LANES,), jnp.float32)
        one_hot = jnp.where(lax.iota(jnp.int32, LANES) == 0, 1.0, 0.0)
        _fill_rows(zbuf, zero_v)
        _fill_rows(ones, one_hot)
        _fill_rows(ones_t, one_hot)

        dsts = (d0, d1, d2)
        outs = (o0, o1, o2)

        @pl.when(cid == 0)
        def _():
            _deg_job(dsts[0], outs[0], acc, zbuf, ones, ones_t, idx_d, idx_dt,
                     sems, sid, plan, n)

        @pl.when(cid == 1)
        def _():
            for ei in (1, 2):
                _deg_job(dsts[ei], outs[ei], acc, zbuf, ones, ones_t, idx_d,
                         idx_dt, sems, sid, plan, n)

    return deg_kernel


def _tc_combine(qs, degs, Ws, bs, n):
    bm = 2000
    grid = (n // bm,)
    d_out = Ws[0].shape[1]

    def body(*refs):
        q_refs = refs[:12]
        deg_refs = refs[12:15]
        w_refs = refs[15:18]
        b_refs = refs[18:21]
        out_ref = refs[21]
        for e in range(3):
            deg = deg_refs[e][:, 0:1]
            cnt = jnp.maximum(deg, 1.0)
            xm = jnp.concatenate([q_refs[e * NQ + q][...] for q in range(NQ)],
                                 axis=1) / cnt
            h = lax.dot_general(xm, w_refs[e][...], (((1,), (0,)), ((), ())),
                                preferred_element_type=jnp.float32,
                                precision=lax.Precision.HIGHEST)
            h = h + b_refs[e][...] * (deg > 0).astype(jnp.float32)
            out_ref[:, e, :] = h

    in_specs = (
        [pl.BlockSpec((bm, QW), lambda i: (i, 0)) for _ in range(12)]
        + [pl.BlockSpec((bm, LANES), lambda i: (i, 0)) for _ in range(3)]
        + [pl.BlockSpec(W.shape, lambda i: (0, 0)) for W in Ws]
        + [pl.BlockSpec((1, d_out), lambda i: (0, 0)) for _ in bs]
    )
    out_spec = pl.BlockSpec((bm, 3, d_out), lambda i: (i, 0, 0))
    return pl.pallas_call(
        body,
        grid=grid,
        in_specs=in_specs,
        out_specs=out_spec,
        out_shape=jax.ShapeDtypeStruct((n, 3, d_out), jnp.float32),
    )(*qs, *degs, *Ws, *[b.reshape(1, -1) for b in bs])


def kernel(x, edge_index_e0, edge_index_e1, edge_index_e2,
           W_e0, b_e0, W_e1, b_e1, W_e2, b_e2):
    n = x.shape[0]
    e = edge_index_e0.shape[1]

    xqs = [x[:, q * QW:(q + 1) * QW] for q in range(NQ)]
    srcs = [ei[0] for ei in (edge_index_e0, edge_index_e1, edge_index_e2)]
    dsts = [ei[1] for ei in (edge_index_e0, edge_index_e1, edge_index_e2)]

    seg = _make_seg_kernel(n, e)(srcs[0], dsts[0], srcs[1], dsts[1],
                                 srcs[2], dsts[2], *xqs)
    degs = _make_deg_kernel(n, e)(*dsts)

    return _tc_combine(seg, degs, (W_e0, W_e1, W_e2), (b_e0, b_e1, b_e2), n)


# speedup vs baseline: 2.5915x; 1.2202x over previous
import functools

import jax
import jax.numpy as jnp
from jax import lax
from jax.experimental import pallas as pl
from jax.experimental.pallas import tpu as pltpu
from jax.experimental.pallas import tpu_sc as plsc

NC = 2
NS = 16
LANES = 16

WIN = 128
QW = 32
NQ = 4

_mesh = plsc.VectorSubcoreMesh(core_axis_name="c", subcore_axis_name="s")
_sc_params = pltpu.CompilerParams(use_tc_tiling_on_sc=False)


def _plan(n, e_pad):
    stripe = ((n + NS - 1) // NS + 7) // 8 * 8
    n_pad = stripe * NS
    nwin = e_pad // (NS * WIN)
    return stripe, n_pad, nwin


def _zero_acc(acc, zbuf, sem, sid, stripe):
    base = sid * stripe
    n_chunks = stripe // WIN
    tail = stripe - n_chunks * WIN
    copies = []
    for i in range(n_chunks):
        copies.append(pltpu.make_async_copy(
            zbuf, acc.at[pl.ds(base + i * WIN, WIN)], sem))
    if tail:
        copies.append(pltpu.make_async_copy(
            zbuf.at[pl.ds(0, tail)], acc.at[pl.ds(base + n_chunks * WIN, tail)], sem))
    for c in copies:
        c.start()
    for c in copies:
        c.wait()


def _write_out(acc, out_hbm, sid, stripe, n):
    last_rows = n - (NS - 1) * stripe

    @pl.when(sid < NS - 1)
    def _():
        pltpu.sync_copy(acc.at[pl.ds(sid * stripe, stripe)],
                        out_hbm.at[pl.ds(sid * stripe, stripe)])

    @pl.when(sid == NS - 1)
    def _():
        pltpu.sync_copy(acc.at[pl.ds((NS - 1) * stripe, last_rows)],
                        out_hbm.at[pl.ds((NS - 1) * stripe, last_rows)])


def _seg_job(src_hbm, dst_hbm, xq_hbm, out_hbm, acc, zbuf, idx_s, idx_d,
             rows, lsem_s, lsem_d, gsem, sid, plan, n):
    stripe, _, nwin = plan
    _zero_acc(acc, zbuf, lsem_s[0], sid, stripe)
    plsc.subcore_barrier()

    def base_of(k):
        return pl.multiple_of((sid + k * NS) * WIN, WIN)

    def load(k, b):
        pltpu.make_async_copy(src_hbm.at[pl.ds(base_of(k), WIN)],
                              idx_s[b], lsem_s[b]).start()
        pltpu.make_async_copy(dst_hbm.at[pl.ds(base_of(k), WIN)],
                              idx_d[b], lsem_d[b]).start()

    def wait_load(k, b):
        pltpu.make_async_copy(src_hbm.at[pl.ds(base_of(k), WIN)],
                              idx_s[b], lsem_s[b]).wait()
        pltpu.make_async_copy(dst_hbm.at[pl.ds(base_of(k), WIN)],
                              idx_d[b], lsem_d[b]).wait()

    def gather(b):
        pltpu.make_async_copy(xq_hbm.at[idx_s[b]], rows[b], gsem[b]).start()

    def wait_gather(b):
        pltpu.make_async_copy(xq_hbm.at[idx_s[b]], rows[b], gsem[b]).wait()

    def scatter(b):
        pltpu.sync_copy(rows[b], acc.at[idx_d[b]], add=True)

    load(0, 0)
    load(1, 1)
    wait_load(0, 0)
    gather(0)

    @pl.loop(0, nwin // 2 - 1)
    def _(j):
        k = 2 * j
        wait_load(k + 1, 1)
        wait_gather(0)
        gather(1)
        scatter(0)
        load(k + 2, 0)
        wait_gather(1)
        wait_load(k + 2, 0)
        gather(0)
        scatter(1)
        load(k + 3, 1)

    wait_load(nwin - 1, 1)
    wait_gather(0)
    gather(1)
    scatter(0)
    wait_gather(1)
    scatter(1)

    plsc.subcore_barrier()
    _write_out(acc, out_hbm, sid, stripe, n)
    plsc.subcore_barrier()


def _deg_job(dst_hbm, out_hbm, acc, zbuf, ones, idx_d, lsem_d, sid, plan, n):
    stripe, _, nwin = plan
    _zero_acc(acc, zbuf, lsem_d[0], sid, stripe)
    plsc.subcore_barrier()

    def base_of(k):
        return pl.multiple_of((sid + k * NS) * WIN, WIN)

    def load(k, b):
        pltpu.make_async_copy(dst_hbm.at[pl.ds(base_of(k), WIN)],
                              idx_d[b], lsem_d[b]).start()

    def wait_load(k, b):
        pltpu.make_async_copy(dst_hbm.at[pl.ds(base_of(k), WIN)],
                              idx_d[b], lsem_d[b]).wait()

    def scatter(b):
        pltpu.sync_copy(ones, acc.at[idx_d[b]], add=True)

    load(0, 0)
    load(1, 1)

    @pl.loop(0, nwin // 2 - 1)
    def _(j):
        k = 2 * j
        wait_load(k, 0)
        scatter(0)
        load(k + 2, 0)
        wait_load(k + 1, 1)
        scatter(1)
        load(k + 3, 1)

    wait_load(nwin - 2, 0)
    scatter(0)
    wait_load(nwin - 1, 1)
    scatter(1)

    plsc.subcore_barrier()
    _write_out(acc, out_hbm, sid, stripe, n)
    plsc.subcore_barrier()


def _fill_rows(ref, vec):
    nrows, width = ref.shape

    @pl.loop(0, nrows)
    def _(i):
        for j in range(width // LANES):
            ref[i, pl.ds(j * LANES, LANES)] = vec


def _make_sc_kernel(n, e_pad):
    plan = _plan(n, e_pad)
    n_pad = plan[1]
    out_t = ([jax.ShapeDtypeStruct((n, QW), jnp.float32) for _ in range(3 * NQ)]
             + [jax.ShapeDtypeStruct((n, QW), jnp.float32) for _ in range(3)])
    scratch = [
        pltpu.VMEM_SHARED((n_pad, QW), jnp.float32),
        pltpu.VMEM((WIN, QW), jnp.float32),
        pltpu.VMEM((WIN, QW), jnp.float32),
        pltpu.VMEM((WIN,), jnp.int32),
        pltpu.VMEM((WIN,), jnp.int32),
        pltpu.VMEM((WIN,), jnp.int32),
        pltpu.VMEM((WIN,), jnp.int32),
        pltpu.VMEM((WIN, QW), jnp.float32),
        pltpu.VMEM((WIN, QW), jnp.float32),
        pltpu.SemaphoreType.DMA,
        pltpu.SemaphoreType.DMA,
        pltpu.SemaphoreType.DMA,
        pltpu.SemaphoreType.DMA,
        pltpu.SemaphoreType.DMA,
        pltpu.SemaphoreType.DMA,
    ]

    @functools.partial(pl.kernel, out_type=out_t, mesh=_mesh,
                       scratch_types=scratch, compiler_params=_sc_params)
    def sc_kernel(s0, d0, s1, d1, s2, d2, x0, x1, x2, x3, *rest):
        outs = rest[:15]
        acc, zbuf, ones = rest[15:18]
        idx_s = rest[18:20]
        idx_d = rest[20:22]
        rows = rest[22:24]
        lsem_s = rest[24:26]
        lsem_d = rest[26:28]
        gsem = rest[28:30]
        cid = lax.axis_index("c")
        sid = lax.axis_index("s")
        zero_v = jnp.zeros((LANES,), jnp.float32)
        one_hot = jnp.where(lax.iota(jnp.int32, LANES) == 0, 1.0, 0.0)
        _fill_rows(zbuf, zero_v)
        _fill_rows(ones, zero_v)

        @pl.loop(0, WIN)
        def _(i):
            ones[i, pl.ds(0, LANES)] = one_hot

        srcs = (s0, s1, s2)
        dsts = (d0, d1, d2)
        xqs = (x0, x1, x2, x3)

        def seg(ei, q):
            _seg_job(srcs[ei], dsts[ei], xqs[q], outs[ei * NQ + q],
                     acc, zbuf, idx_s, idx_d, rows, lsem_s, lsem_d, gsem,
                     sid, plan, n)

        def deg(ei):
            _deg_job(dsts[ei], outs[12 + ei], acc, zbuf, ones, idx_d,
                     lsem_d, sid, plan, n)

        @pl.when(cid == 0)
        def _():
            for ei in range(3):
                seg(ei, 0)
                seg(ei, 1)
            deg(0)
            deg(1)

        @pl.when(cid == 1)
        def _():
            for ei in range(3):
                seg(ei, 2)
                seg(ei, 3)
            deg(2)

    return sc_kernel


def _tc_combine(qs, degs, Ws, bs, n):
    bm = 2000
    grid = (n // bm,)
    d_out = Ws[0].shape[1]

    def body(*refs):
        q_refs = refs[:12]
        deg_refs = refs[12:15]
        w_refs = refs[15:18]
        b_refs = refs[18:21]
        out_ref = refs[21]
        for e in range(3):
            deg = deg_refs[e][:, 0:1]
            cnt = jnp.maximum(deg, 1.0)
            xm = jnp.concatenate([q_refs[e * NQ + q][...] for q in range(NQ)],
                                 axis=1) / cnt
            h = lax.dot_general(xm, w_refs[e][...], (((1,), (0,)), ((), ())),
                                preferred_element_type=jnp.float32,
                                precision=lax.Precision.HIGHEST)
            h = h + b_refs[e][...] * (deg > 0).astype(jnp.float32)
            out_ref[:, e, :] = h

    in_specs = (
        [pl.BlockSpec((bm, QW), lambda i: (i, 0)) for _ in range(12)]
        + [pl.BlockSpec((bm, QW), lambda i: (i, 0)) for _ in range(3)]
        + [pl.BlockSpec(W.shape, lambda i: (0, 0)) for W in Ws]
        + [pl.BlockSpec((1, d_out), lambda i: (0, 0)) for _ in bs]
    )
    out_spec = pl.BlockSpec((bm, 3, d_out), lambda i: (i, 0, 0))
    return pl.pallas_call(
        body,
        grid=grid,
        in_specs=in_specs,
        out_specs=out_spec,
        out_shape=jax.ShapeDtypeStruct((n, 3, d_out), jnp.float32),
    )(*qs, *degs, *Ws, *[b.reshape(1, -1) for b in bs])


def kernel(x, edge_index_e0, edge_index_e1, edge_index_e2,
           W_e0, b_e0, W_e1, b_e1, W_e2, b_e2):
    n = x.shape[0]
    e = edge_index_e0.shape[1]

    group = NS * WIN * 2
    e_pad = (e + group - 1) // group * group
    pad = e_pad - e
    n_pad = _plan(n, e_pad)[1]

    xqs = [x[:, q * QW:(q + 1) * QW] for q in range(NQ)]
    pad_src = (jnp.arange(pad, dtype=jnp.int32) * 61) % n
    pad_dst = n + (jnp.arange(pad, dtype=jnp.int32) % (n_pad - n))
    srcs = [jnp.concatenate([ei[0], pad_src])
            for ei in (edge_index_e0, edge_index_e1, edge_index_e2)]
    dsts = [jnp.concatenate([ei[1], pad_dst])
            for ei in (edge_index_e0, edge_index_e1, edge_index_e2)]

    sc_out = _make_sc_kernel(n, e_pad)(srcs[0], dsts[0], srcs[1], dsts[1],
                                       srcs[2], dsts[2], *xqs)
    seg, degs = sc_out[:12], sc_out[12:]

    return _tc_combine(seg, degs, (W_e0, W_e1, W_e2), (b_e0, b_e1, b_e2), n)


# speedup vs baseline: 3.1408x; 1.2119x over previous
import functools

import jax
import jax.numpy as jnp
from jax import lax
from jax.experimental import pallas as pl
from jax.experimental.pallas import tpu as pltpu
from jax.experimental.pallas import tpu_sc as plsc

NC = 2
NS = 16
LANES = 16

WIN = 128
QW = 32
NQ = 4

_mesh = plsc.VectorSubcoreMesh(core_axis_name="c", subcore_axis_name="s")
_sc_params = pltpu.CompilerParams(use_tc_tiling_on_sc=False)


def _plan(n, e_pad):
    stripe = ((n + NS - 1) // NS + 7) // 8 * 8
    n_pad = stripe * NS
    nwin = e_pad // (NS * WIN)
    return stripe, n_pad, nwin


def _zero_acc(acc, zbuf, sem, sid, stripe):
    base = sid * stripe
    n_chunks = stripe // WIN
    tail = stripe - n_chunks * WIN
    copies = []
    for i in range(n_chunks):
        copies.append(pltpu.make_async_copy(
            zbuf, acc.at[pl.ds(base + i * WIN, WIN)], sem))
    if tail:
        copies.append(pltpu.make_async_copy(
            zbuf.at[pl.ds(0, tail)], acc.at[pl.ds(base + n_chunks * WIN, tail)], sem))
    for c in copies:
        c.start()
    for c in copies:
        c.wait()


def _write_out(acc, out_hbm, sid, stripe, n, col0=None):
    last_rows = n - (NS - 1) * stripe

    def dst(r0, nr):
        if col0 is None:
            return out_hbm.at[pl.ds(r0, nr)]
        return out_hbm.at[pl.ds(r0, nr), pl.ds(col0, QW)]

    @pl.when(sid < NS - 1)
    def _():
        pltpu.sync_copy(acc.at[pl.ds(sid * stripe, stripe)],
                        dst(sid * stripe, stripe))

    @pl.when(sid == NS - 1)
    def _():
        pltpu.sync_copy(acc.at[pl.ds((NS - 1) * stripe, last_rows)],
                        dst((NS - 1) * stripe, last_rows))


def _seg_job(src_hbm, dst_hbm, xq_hbm, out_hbm, acc, zbuf, idx_s, idx_d,
             rows, lsem_s, lsem_d, gsem, sid, plan, n, col0):
    stripe, _, nwin = plan
    _zero_acc(acc, zbuf, lsem_s[0], sid, stripe)
    plsc.subcore_barrier()

    def base_of(k):
        return pl.multiple_of((sid + k * NS) * WIN, WIN)

    def load(k, b):
        pltpu.make_async_copy(src_hbm.at[pl.ds(base_of(k), WIN)],
                              idx_s[b], lsem_s[b]).start()
        pltpu.make_async_copy(dst_hbm.at[pl.ds(base_of(k), WIN)],
                              idx_d[b], lsem_d[b]).start()

    def wait_load(k, b):
        pltpu.make_async_copy(src_hbm.at[pl.ds(base_of(k), WIN)],
                              idx_s[b], lsem_s[b]).wait()
        pltpu.make_async_copy(dst_hbm.at[pl.ds(base_of(k), WIN)],
                              idx_d[b], lsem_d[b]).wait()

    def gather(b):
        pltpu.make_async_copy(xq_hbm.at[idx_s[b]], rows[b], gsem[b]).start()

    def wait_gather(b):
        pltpu.make_async_copy(xq_hbm.at[idx_s[b]], rows[b], gsem[b]).wait()

    def scatter(b):
        pltpu.sync_copy(rows[b], acc.at[idx_d[b]], add=True)

    load(0, 0)
    load(1, 1)
    wait_load(0, 0)
    gather(0)

    @pl.loop(0, nwin // 2 - 1)
    def _(j):
        k = 2 * j
        wait_load(k + 1, 1)
        wait_gather(0)
        gather(1)
        scatter(0)
        load(k + 2, 0)
        wait_gather(1)
        wait_load(k + 2, 0)
        gather(0)
        scatter(1)
        load(k + 3, 1)

    wait_load(nwin - 1, 1)
    wait_gather(0)
    gather(1)
    scatter(0)
    wait_gather(1)
    scatter(1)

    plsc.subcore_barrier()
    _write_out(acc, out_hbm, sid, stripe, n, col0)
    plsc.subcore_barrier()


def _deg_job(dst_hbm, out_hbm, acc, zbuf, ones, idx_d, lsem_d, sid, plan, n):
    stripe, _, nwin = plan
    _zero_acc(acc, zbuf, lsem_d[0], sid, stripe)
    plsc.subcore_barrier()

    def base_of(k):
        return pl.multiple_of((sid + k * NS) * WIN, WIN)

    def load(k, b):
        pltpu.make_async_copy(dst_hbm.at[pl.ds(base_of(k), WIN)],
                              idx_d[b], lsem_d[b]).start()

    def wait_load(k, b):
        pltpu.make_async_copy(dst_hbm.at[pl.ds(base_of(k), WIN)],
                              idx_d[b], lsem_d[b]).wait()

    def scatter(b):
        pltpu.sync_copy(ones, acc.at[idx_d[b]], add=True)

    load(0, 0)
    load(1, 1)

    @pl.loop(0, nwin // 2 - 1)
    def _(j):
        k = 2 * j
        wait_load(k, 0)
        scatter(0)
        load(k + 2, 0)
        wait_load(k + 1, 1)
        scatter(1)
        load(k + 3, 1)

    wait_load(nwin - 2, 0)
    scatter(0)
    wait_load(nwin - 1, 1)
    scatter(1)

    plsc.subcore_barrier()
    _write_out(acc, out_hbm, sid, stripe, n)
    plsc.subcore_barrier()


def _fill_rows(ref, vec):
    nrows, width = ref.shape

    @pl.loop(0, nrows)
    def _(i):
        for j in range(width // LANES):
            ref[i, pl.ds(j * LANES, LANES)] = vec


def _make_sc_kernel(n, e_pad):
    plan = _plan(n, e_pad)
    n_pad = plan[1]
    out_t = ([jax.ShapeDtypeStruct((n, NQ * QW), jnp.float32) for _ in range(3)]
             + [jax.ShapeDtypeStruct((n, QW), jnp.float32) for _ in range(3)])
    scratch = [
        pltpu.VMEM_SHARED((n_pad, QW), jnp.float32),
        pltpu.VMEM((WIN, QW), jnp.float32),
        pltpu.VMEM((WIN, QW), jnp.float32),
        pltpu.VMEM((WIN,), jnp.int32),
        pltpu.VMEM((WIN,), jnp.int32),
        pltpu.VMEM((WIN,), jnp.int32),
        pltpu.VMEM((WIN,), jnp.int32),
        pltpu.VMEM((WIN, QW), jnp.float32),
        pltpu.VMEM((WIN, QW), jnp.float32),
        pltpu.SemaphoreType.DMA,
        pltpu.SemaphoreType.DMA,
        pltpu.SemaphoreType.DMA,
        pltpu.SemaphoreType.DMA,
        pltpu.SemaphoreType.DMA,
        pltpu.SemaphoreType.DMA,
    ]

    @functools.partial(pl.kernel, out_type=out_t, mesh=_mesh,
                       scratch_types=scratch, compiler_params=_sc_params)
    def sc_kernel(s0, d0, s1, d1, s2, d2, x0, x1, x2, x3, *rest):
        outs = rest[:6]
        acc, zbuf, ones = rest[6:9]
        idx_s = rest[9:11]
        idx_d = rest[11:13]
        rows = rest[13:15]
        lsem_s = rest[15:17]
        lsem_d = rest[17:19]
        gsem = rest[19:21]
        cid = lax.axis_index("c")
        sid = lax.axis_index("s")
        zero_v = jnp.zeros((LANES,), jnp.float32)
        one_hot = jnp.where(lax.iota(jnp.int32, LANES) == 0, 1.0, 0.0)
        _fill_rows(zbuf, zero_v)
        _fill_rows(ones, zero_v)

        @pl.loop(0, WIN)
        def _(i):
            ones[i, pl.ds(0, LANES)] = one_hot

        srcs = (s0, s1, s2)
        dsts = (d0, d1, d2)
        xqs = (x0, x1, x2, x3)

        def seg(ei, q):
            _seg_job(srcs[ei], dsts[ei], xqs[q], outs[ei],
                     acc, zbuf, idx_s, idx_d, rows, lsem_s, lsem_d, gsem,
                     sid, plan, n, q * QW)

        def deg(ei):
            _deg_job(dsts[ei], outs[3 + ei], acc, zbuf, ones, idx_d,
                     lsem_d, sid, plan, n)

        @pl.when(cid == 0)
        def _():
            for ei in range(3):
                seg(ei, 0)
                seg(ei, 1)
            deg(0)
            deg(1)

        @pl.when(cid == 1)
        def _():
            for ei in range(3):
                seg(ei, 2)
                seg(ei, 3)
            deg(2)

    return sc_kernel


def _tc_combine(qs, degs, Ws, bs, n):
    bm = 2000
    grid = (n // bm,)
    d_out = Ws[0].shape[1]

    def body(*refs):
        seg_refs = refs[:3]
        deg_refs = refs[3:6]
        w_refs = refs[6:9]
        b_refs = refs[9:12]
        out_ref = refs[12]
        for e in range(3):
            deg = deg_refs[e][:, 0:1]
            cnt = jnp.maximum(deg, 1.0)
            xm = seg_refs[e][...] / cnt
            h = lax.dot_general(xm, w_refs[e][...], (((1,), (0,)), ((), ())),
                                preferred_element_type=jnp.float32)
            h = h + b_refs[e][...] * (deg > 0).astype(jnp.float32)
            out_ref[:, e, :] = h

    in_specs = (
        [pl.BlockSpec((bm, NQ * QW), lambda i: (i, 0)) for _ in range(3)]
        + [pl.BlockSpec((bm, QW), lambda i: (i, 0)) for _ in range(3)]
        + [pl.BlockSpec(W.shape, lambda i: (0, 0)) for W in Ws]
        + [pl.BlockSpec((1, d_out), lambda i: (0, 0)) for _ in bs]
    )
    out_spec = pl.BlockSpec((bm, 3, d_out), lambda i: (i, 0, 0))
    return pl.pallas_call(
        body,
        grid=grid,
        in_specs=in_specs,
        out_specs=out_spec,
        out_shape=jax.ShapeDtypeStruct((n, 3, d_out), jnp.float32),
    )(*qs, *degs, *Ws, *[b.reshape(1, -1) for b in bs])


def kernel(x, edge_index_e0, edge_index_e1, edge_index_e2,
           W_e0, b_e0, W_e1, b_e1, W_e2, b_e2):
    n = x.shape[0]
    e = edge_index_e0.shape[1]

    group = NS * WIN * 2
    e_pad = (e + group - 1) // group * group
    pad = e_pad - e
    n_pad = _plan(n, e_pad)[1]

    xqs = [x[:, q * QW:(q + 1) * QW] for q in range(NQ)]
    pad_src = (jnp.arange(pad, dtype=jnp.int32) * 61) % n
    pad_dst = n + (jnp.arange(pad, dtype=jnp.int32) % (n_pad - n))
    srcs = [jnp.concatenate([ei[0], pad_src])
            for ei in (edge_index_e0, edge_index_e1, edge_index_e2)]
    dsts = [jnp.concatenate([ei[1], pad_dst])
            for ei in (edge_index_e0, edge_index_e1, edge_index_e2)]

    sc_out = _make_sc_kernel(n, e_pad)(srcs[0], dsts[0], srcs[1], dsts[1],
                                       srcs[2], dsts[2], *xqs)
    seg, degs = sc_out[:3], sc_out[3:]

    return _tc_combine(seg, degs, (W_e0, W_e1, W_e2), (b_e0, b_e1, b_e2), n)


# speedup vs baseline: 5.2462x; 1.6703x over previous
import functools

import jax
import jax.numpy as jnp
from jax import lax
from jax.experimental import pallas as pl
from jax.experimental.pallas import tpu as pltpu
from jax.experimental.pallas import tpu_sc as plsc

NC = 2
NS = 16
LANES = 16

WIN = 128
UNIT = 3
QW = 32
NQ = 4

_mesh = plsc.VectorSubcoreMesh(core_axis_name="c", subcore_axis_name="s")
_sc_params = pltpu.CompilerParams(use_tc_tiling_on_sc=False)


def _plan(n, e_pad):
    stripe = ((n + NS - 1) // NS + 7) // 8 * 8
    n_pad = stripe * NS
    nwin = e_pad // (NS * WIN * UNIT)
    return stripe, n_pad, nwin


def _zero_acc(acc, zbuf, sem, sid, stripe):
    base = sid * stripe
    n_chunks = stripe // WIN
    tail = stripe - n_chunks * WIN
    copies = []
    for i in range(n_chunks):
        copies.append(pltpu.make_async_copy(
            zbuf, acc.at[pl.ds(base + i * WIN, WIN)], sem))
    if tail:
        copies.append(pltpu.make_async_copy(
            zbuf.at[pl.ds(0, tail)], acc.at[pl.ds(base + n_chunks * WIN, tail)], sem))
    for c in copies:
        c.start()
    for c in copies:
        c.wait()


def _flush(acc, out_hbm, aux, sid, stripe, n, col0, wsem, zsem, obase=0):
    last_rows = n - (NS - 1) * stripe

    CH = 1024

    def wcopy(r0, c):
        return pltpu.make_async_copy(
            acc.at[pl.ds(r0, c)],
            out_hbm.at[pl.ds(obase + r0, c), pl.ds(col0, QW)], wsem)

    def zcopy(r0, c):
        return pltpu.make_async_copy(
            aux.at[pl.ds(0, c)], acc.at[pl.ds(r0, c)], zsem)

    def emit(base, nrows):
        nw = nrows // CH
        wtail = nrows - nw * CH

        @pl.loop(0, nw)
        def _(j):
            wcopy(base + j * CH, CH).start()
        if wtail:
            wcopy(base + nw * CH, wtail).start()

        @pl.loop(0, nw)
        def _(j):
            wcopy(base + j * CH, CH).wait()

            @pl.loop(0, CH // WIN)
            def _(i):
                zcopy(base + j * CH + i * WIN, WIN).start()

        if wtail:
            wcopy(base + nw * CH, wtail).wait()
            zcopy(base + nw * CH, wtail).start()

        @pl.loop(0, nw * (CH // WIN))
        def _(i):
            zcopy(base, WIN).wait()
        if wtail:
            zcopy(base + nw * CH, wtail).wait()

    @pl.when(sid < NS - 1)
    def _():
        emit(sid * stripe, stripe)

    @pl.when(sid == NS - 1)
    def _():
        emit((NS - 1) * stripe, last_rows)


def _seg_job(src_hbm, dst_hbm, xq_hbm, out_hbm, acc, zbuf, idx_s, idx_d,
             rows, lsem_s, lsem_d, gsem, sid, plan, n, col0, ebase, obase,
             dbase):
    stripe, _, nwin = plan

    def row0_of(k):
        return pl.multiple_of((sid + k * NS) * UNIT, UNIT)

    def load(k, b):
        pltpu.make_async_copy(src_hbm.at[pl.ds(ebase + row0_of(k), UNIT)],
                              idx_s[b], lsem_s[b]).start()
        pltpu.make_async_copy(dst_hbm.at[pl.ds(dbase + row0_of(k), UNIT)],
                              idx_d[b], lsem_d[b]).start()

    def wait_load(k, b):
        pltpu.make_async_copy(src_hbm.at[pl.ds(ebase + row0_of(k), UNIT)],
                              idx_s[b], lsem_s[b]).wait()
        pltpu.make_async_copy(dst_hbm.at[pl.ds(dbase + row0_of(k), UNIT)],
                              idx_d[b], lsem_d[b]).wait()

    def gather(b):
        for w in range(UNIT):
            pltpu.make_async_copy(xq_hbm.at[idx_s[b].at[w]],
                                  rows[b].at[pl.ds(w * WIN, WIN)],
                                  gsem[b]).start()

    def wait_gather(b):
        for w in range(UNIT):
            pltpu.make_async_copy(xq_hbm.at[idx_s[b].at[w]],
                                  rows[b].at[pl.ds(w * WIN, WIN)],
                                  gsem[b]).wait()

    def scatter(b):
        for w in range(UNIT):
            pltpu.sync_copy(rows[b].at[pl.ds(w * WIN, WIN)],
                            acc.at[idx_d[b].at[w]], add=True)

    load(0, 0)
    load(1, 1)
    wait_load(0, 0)
    gather(0)

    @pl.loop(0, nwin // 2 - 1)
    def _(j):
        k = 2 * j
        wait_load(k + 1, 1)
        wait_gather(0)
        gather(1)
        scatter(0)
        load(k + 2, 0)
        wait_gather(1)
        wait_load(k + 2, 0)
        gather(0)
        scatter(1)
        load(k + 3, 1)

    wait_load(nwin - 1, 1)
    wait_gather(0)
    gather(1)
    scatter(0)
    wait_gather(1)
    scatter(1)

    plsc.subcore_barrier()
    _flush(acc, out_hbm, zbuf, sid, stripe, n, col0, lsem_s[0], lsem_s[1],
           obase)
    plsc.subcore_barrier()


def _deg_job(dst_hbm, out_hbm, acc, aux, idx_d, lsem_d, sid, plan, n,
             col0, k0, nunits, ebase):
    stripe, _, _ = plan
    one_hot = jnp.where(lax.iota(jnp.int32, LANES) == 0, 1.0, 0.0)

    @pl.loop(0, WIN)
    def _(i):
        aux[i, pl.ds(0, LANES)] = one_hot

    def row0_of(k):
        return ebase + pl.multiple_of((sid + k * NS) * UNIT, UNIT)

    def load(k, b):
        pltpu.make_async_copy(dst_hbm.at[pl.ds(row0_of(k), UNIT)],
                              idx_d[b], lsem_d[b]).start()

    def wait_load(k, b):
        pltpu.make_async_copy(dst_hbm.at[pl.ds(row0_of(k), UNIT)],
                              idx_d[b], lsem_d[b]).wait()

    def scatter(b):
        for w in range(UNIT):
            pltpu.sync_copy(aux, acc.at[idx_d[b].at[w]], add=True)

    load(k0, 0)
    load(k0 + 1, 1)

    @pl.loop(0, nunits // 2 - 1)
    def _(j):
        k = k0 + 2 * j
        wait_load(k, 0)
        scatter(0)
        load(k + 2, 0)
        wait_load(k + 1, 1)
        scatter(1)
        load(k + 3, 1)

    wait_load(k0 + nunits - 2, 0)
    scatter(0)
    wait_load(k0 + nunits - 1, 1)
    scatter(1)

    _fill_rows(aux, jnp.zeros((LANES,), jnp.float32))
    plsc.subcore_barrier()
    _flush(acc, out_hbm, aux, sid, stripe, n, col0, lsem_d[0], lsem_d[1])
    plsc.subcore_barrier()


def _fill_rows(ref, vec):
    nrows, width = ref.shape

    @pl.loop(0, nrows)
    def _(i):
        for j in range(width // LANES):
            ref[i, pl.ds(j * LANES, LANES)] = vec


def _make_sc_kernel(n, e_pad):
    plan = _plan(n, e_pad)
    n_pad = plan[1]
    out_t = [jax.ShapeDtypeStruct((3 * n, NQ * QW), jnp.float32),
             jax.ShapeDtypeStruct((n, NQ * QW), jnp.float32)]
    scratch = [
        pltpu.VMEM_SHARED((n_pad, QW), jnp.float32),
        pltpu.VMEM((WIN, QW), jnp.float32),
        pltpu.VMEM((UNIT, WIN), jnp.int32),
        pltpu.VMEM((UNIT, WIN), jnp.int32),
        pltpu.VMEM((UNIT, WIN), jnp.int32),
        pltpu.VMEM((UNIT, WIN), jnp.int32),
        pltpu.VMEM((UNIT * WIN, QW), jnp.float32),
        pltpu.VMEM((UNIT * WIN, QW), jnp.float32),
        pltpu.SemaphoreType.DMA,
        pltpu.SemaphoreType.DMA,
        pltpu.SemaphoreType.DMA,
        pltpu.SemaphoreType.DMA,
        pltpu.SemaphoreType.DMA,
        pltpu.SemaphoreType.DMA,
    ]

    erows = e_pad // WIN

    @functools.partial(pl.kernel, out_type=out_t, mesh=_mesh,
                       scratch_types=scratch, compiler_params=_sc_params)
    def sc_kernel(src_all, dst_all, xr, seg_out, deg_out, *rest):
        acc, aux = rest[0:2]
        idx_s = rest[2:4]
        idx_d = rest[4:6]
        rows = rest[6:8]
        lsem_s = rest[8:10]
        lsem_d = rest[10:12]
        gsem = rest[12:14]
        cid = lax.axis_index("c")
        sid = lax.axis_index("s")
        zero_v = jnp.zeros((LANES,), jnp.float32)
        _fill_rows(aux, zero_v)
        _zero_acc(acc, aux, lsem_s[0], sid, plan[0])
        plsc.subcore_barrier()

        nwin = plan[2]
        mid = ((nwin // 2) + 1) // 2 * 2

        def seg(q0):
            @pl.loop(0, 3)
            def _(ei):
                for q in (q0, q0 + 1):
                    _seg_job(src_all, dst_all, xr, seg_out,
                             acc, aux, idx_s, idx_d, rows, lsem_s, lsem_d,
                             gsem, sid, plan, n, q * QW,
                             (ei * NQ + q) * erows, ei * n,
                             dbase=ei * erows)

        def deg(ei, col0, k0, nunits):
            _deg_job(dst_all, deg_out, acc, aux, idx_d,
                     lsem_d, sid, plan, n, col0, k0, nunits, ei * erows)

        @pl.when(cid == 0)
        def _():
            seg(0)
            deg(0, 0, 0, nwin)
            deg(1, QW, 0, mid)

        @pl.when(cid == 1)
        def _():
            seg(2)
            deg(2, 2 * QW, 0, nwin)
            deg(1, 3 * QW, mid, nwin - mid)

    return sc_kernel


def _tc_combine(qs, degs, Ws, bs, n):
    bm = 2000
    grid = (n // bm,)
    d_out = Ws[0].shape[1]

    def body(*refs):
        seg_refs = refs[:3]
        deg_ref = refs[3]
        w_refs = refs[4:7]
        b_refs = refs[7:10]
        out_ref = refs[10]
        for e in range(3):
            deg = deg_ref[:, e * QW:e * QW + 1]
            if e == 1:
                deg = deg + deg_ref[:, 3 * QW:3 * QW + 1]
            cnt = jnp.maximum(deg, 1.0)
            xm = seg_refs[e][...] / cnt
            h = lax.dot_general(xm, w_refs[e][...], (((1,), (0,)), ((), ())),
                                preferred_element_type=jnp.float32)
            h = h + b_refs[e][...] * (deg > 0).astype(jnp.float32)
            out_ref[:, e, :] = h

    nb = n // bm
    in_specs = (
        [pl.BlockSpec((bm, NQ * QW), lambda i, e=e: (i + e * nb, 0))
         for e in range(3)]
        + [pl.BlockSpec((bm, NQ * QW), lambda i: (i, 0))]
        + [pl.BlockSpec(W.shape, lambda i: (0, 0)) for W in Ws]
        + [pl.BlockSpec((1, d_out), lambda i: (0, 0)) for _ in bs]
    )
    out_spec = pl.BlockSpec((bm, 3, d_out), lambda i: (i, 0, 0))
    return pl.pallas_call(
        body,
        grid=grid,
        in_specs=in_specs,
        out_specs=out_spec,
        out_shape=jax.ShapeDtypeStruct((n, 3, d_out), jnp.float32),
    )(qs, qs, qs, degs, *Ws, *[b.reshape(1, -1) for b in bs])


def kernel(x, edge_index_e0, edge_index_e1, edge_index_e2,
           W_e0, b_e0, W_e1, b_e1, W_e2, b_e2):
    n = x.shape[0]
    e = edge_index_e0.shape[1]

    group = NS * WIN * UNIT * 2
    e_pad = (e + group - 1) // group * group
    pad = e_pad - e
    n_pad = _plan(n, e_pad)[1]

    pad_src = (jnp.arange(pad, dtype=jnp.int32) * 61) % n
    pad_dst = n + (jnp.arange(pad, dtype=jnp.int32) % (n_pad - n))
    src_all = jnp.concatenate(
        [(jnp.concatenate([ei[0], pad_src]) * NQ + q)
         .reshape(e_pad // WIN, WIN)
         for ei in (edge_index_e0, edge_index_e1, edge_index_e2)
         for q in range(NQ)])
    dst_all = jnp.concatenate(
        [jnp.concatenate([ei[1], pad_dst]).reshape(e_pad // WIN, WIN)
         for ei in (edge_index_e0, edge_index_e1, edge_index_e2)])
    xr = x.reshape(NQ * n, QW)

    seg, degs = _make_sc_kernel(n, e_pad)(src_all, dst_all, xr)

    return _tc_combine(seg, degs, (W_e0, W_e1, W_e2), (b_e0, b_e1, b_e2), n)
